# Initial kernel scaffold; baseline (speedup 1.0000x reference)
import jax
import jax.numpy as jnp
from jax.experimental import pallas as pl


def kernel(guide_in):
    raise NotImplementedError("write your pallas kernel here")


# speedup vs baseline: 16.2119x; 16.2119x over previous
import functools

import jax
import jax.numpy as jnp
import numpy as np
from jax import lax
from jax.experimental import pallas as pl
from jax.experimental.pallas import tpu as pltpu
from jax.experimental.pallas import tpu_sc as plsc

_H = 224
_W = 224
_V = _H * _W
_E_ROW = (_H - 1) * _W
_E_COL = _H * (_W - 1)
_E = _E_ROW + _E_COL
_CHUNK = 8192
_NCHUNK = -(-_E // _CHUNK)
_E_PAD = _NCHUNK * _CHUNK


def _val_table() -> np.ndarray:
    row_u = np.arange(_E_ROW, dtype=np.int32)
    j = np.arange(_E_COL, dtype=np.int32)
    col_u = (j // (_W - 1)) * _W + (j % (_W - 1))
    return np.concatenate([2 * row_u, 2 * col_u + 1])


def _sload(ref, i):
    return ref[pl.ds(i, 16)][0]


def _sstore(ref, i, val):
    vec = ref[pl.ds(i, 16)]
    lane0 = lax.iota(jnp.int32, 16) == 0
    ref[pl.ds(i, 16)] = jnp.where(lane0, val, vec)


_FIND_LEVELS = 16


def _uf_body(svals_hbm, out_hbm, parent_v, ids_v, out_v):
    c = lax.axis_index("c")
    s = lax.axis_index("s")

    @pl.when(s == 0)
    def _():
        def init(i, carry):
            parent_v[pl.ds(i * 16, 16)] = jnp.full((16,), -1, jnp.int32)
            return carry

        lax.fori_loop(0, (_V + 16) // 16, init, jnp.int32(0))

        def find(x0):
            def walk(level, x, px):
                gp = _sload(parent_v, px)
                if level >= _FIND_LEVELS:
                    return px
                def deeper():
                    _sstore(parent_v, x, gp)
                    return walk(level + 1, px, gp)
                return lax.cond(gp < 0, lambda: px, deeper)

            px0 = _sload(parent_v, x0)
            return lax.cond(px0 < 0, lambda: x0, lambda: walk(0, x0, px0))

        def chunk_body(k, cnt):
            pltpu.sync_copy(svals_hbm.at[c, pl.ds(k * _CHUNK, _CHUNK)],
                            ids_v.at[pl.ds(0, _CHUNK)])

            def edge_body(j, cnt):
                def live(cnt):
                    val = _sload(ids_v, j)
                    u = val >> 1
                    v = u + jnp.where((val & 1) == 1, jnp.int32(1), jnp.int32(_W))
                    ru = find(u)
                    rv = find(v)

                    def take(cnt):
                        su = _sload(parent_v, ru)
                        sv = _sload(parent_v, rv)
                        big_u = su <= sv
                        hi = jnp.where(big_u, ru, rv)
                        lo = jnp.where(big_u, rv, ru)
                        _sstore(parent_v, lo, hi)
                        _sstore(parent_v, hi, su + sv)
                        _sstore(out_v, cnt, val)
                        return cnt + jnp.int32(1)

                    return lax.cond(ru != rv, take, lambda t: t, cnt)

                return lax.cond(cnt < jnp.int32(_V - 1), live, lambda t: t, cnt)

            return lax.fori_loop(0, _CHUNK, edge_body, cnt)

        lax.fori_loop(0, _NCHUNK, chunk_body, jnp.int32(0))
        pltpu.sync_copy(out_v.at[pl.ds(0, _V)], out_hbm.at[c])


@jax.jit
def _run_uf(svals):
    B = svals.shape[0]
    mesh = plsc.VectorSubcoreMesh(core_axis_name="c", subcore_axis_name="s")
    uf = pl.kernel(
        _uf_body,
        out_type=jax.ShapeDtypeStruct((B, _V), jnp.int32),
        mesh=mesh,
        scratch_types=[
            pltpu.VMEM((_V + 16,), jnp.int32),
            pltpu.VMEM((_CHUNK + 16,), jnp.int32),
            pltpu.VMEM((_V + 16,), jnp.int32),
        ],
    )
    return uf(svals)


def kernel(guide_in):
    B, C, H, W = guide_in.shape
    weight_row = jnp.linalg.norm(
        guide_in[:, :, :-1, :] - guide_in[:, :, 1:, :], axis=1).reshape(B, -1)
    weight_col = jnp.linalg.norm(
        guide_in[:, :, :, :-1] - guide_in[:, :, :, 1:], axis=1).reshape(B, -1)
    weight = jnp.concatenate([weight_row, weight_col], axis=1)

    order = jnp.argsort(weight, axis=1, stable=True)
    vals = jnp.asarray(_val_table())
    svals = vals[order]
    svals = jnp.pad(svals, ((0, 0), (0, _E_PAD - _E)))

    out_vals = _run_uf(svals)[:, : _V - 1]
    u = out_vals >> 1
    v = u + jnp.where((out_vals & 1) == 1, jnp.int32(1), jnp.int32(W))
    return jnp.stack([u, v], axis=-1).astype(jnp.int32)
